# Initial kernel scaffold; baseline (speedup 1.0000x reference)
import jax
import jax.numpy as jnp
from jax.experimental import pallas as pl


def kernel(x, edge_index, edge_weight, W_ih0, W_hh0, b_ih0, b_hh0, W_ih1, W_hh1, b_ih1, b_hh1, Wg0, bg0, Wg1, bg1, W_fc1, b_fc1, W_fc2, b_fc2):
    raise NotImplementedError("write your pallas kernel here")


# speedup vs baseline: 1.0001x; 1.0001x over previous
import jax
import jax.numpy as jnp
from jax.experimental import pallas as pl


def _lstm_layer(x_seq, W_ih, W_hh, b_ih, b_hh):
    B = x_seq.shape[0]
    Hh = W_hh.shape[0]
    xs = jnp.transpose(x_seq, (1, 0, 2))

    def step(carry, xt):
        h, c = carry
        gates = xt @ W_ih + h @ W_hh + b_ih + b_hh
        i, f, g, o = jnp.split(gates, 4, axis=-1)
        i = jax.nn.sigmoid(i)
        f = jax.nn.sigmoid(f)
        g = jnp.tanh(g)
        o = jax.nn.sigmoid(o)
        c = f * c + i * g
        h = o * jnp.tanh(c)
        return (h, c), h

    init = (jnp.zeros((B, Hh), x_seq.dtype), jnp.zeros((B, Hh), x_seq.dtype))
    (_, _), ys = jax.lax.scan(step, init, xs)
    return jnp.transpose(ys, (1, 0, 2))


def _gcn_conv(x, src, dst, ew, W, b, n):
    xw = x @ W
    loop = jnp.arange(n, dtype=src.dtype)
    src2 = jnp.concatenate([src, loop])
    dst2 = jnp.concatenate([dst, loop])
    ew2 = jnp.concatenate([ew, jnp.ones((n,), x.dtype)])
    deg = jax.ops.segment_sum(ew2, dst2, num_segments=n)
    dinv = jnp.where(deg > 0, jax.lax.rsqrt(jnp.maximum(deg, 1e-12)), 0.0)
    norm = dinv[src2] * ew2 * dinv[dst2]
    msgs = xw[src2] * norm[:, None]
    out = jax.ops.segment_sum(msgs, dst2, num_segments=n)
    return out + b


def kernel(x, edge_index, edge_weight, W_ih0, W_hh0, b_ih0, b_hh0, W_ih1, W_hh1, b_ih1, b_hh1, Wg0, bg0, Wg1, bg1, W_fc1, b_fc1, W_fc2, b_fc2):
    B, A, Sl, Ff = x.shape
    x_lstm = x.reshape(B * A, Sl, Ff)
    h0 = _lstm_layer(x_lstm, W_ih0, W_hh0, b_ih0, b_hh0)
    h1 = _lstm_layer(h0, W_ih1, W_hh1, b_ih1, b_hh1)
    node_features = h1[:, -1, :]
    n = B * A
    src = edge_index[0]
    dst = edge_index[1]
    node_features = _gcn_conv(node_features, src, dst, edge_weight, Wg0, bg0, n)
    node_features = jax.nn.relu(node_features)
    node_features = _gcn_conv(node_features, src, dst, edge_weight, Wg1, bg1, n)
    out = jax.nn.relu(node_features @ W_fc1 + b_fc1)
    out = out @ W_fc2 + b_fc2
    return out.reshape(B, A, -1)


# speedup vs baseline: 9.7110x; 9.7104x over previous
import functools

import jax
import jax.numpy as jnp
from jax import lax
from jax.experimental import pallas as pl
from jax.experimental.pallas import tpu as pltpu
from jax.experimental.pallas import tpu_sc as plsc

N = 10000
E = 640000
D = 64
NC = 2
NS = 16
NW = NC * NS
L = 16
C = 128
NCHUNK = E // C
TPW = -(-NCHUNK // NW)
RPS = N // NS

_mesh = plsc.VectorSubcoreMesh(
    core_axis_name="c", subcore_axis_name="s", num_cores=NC, num_subcores=NS)


@functools.partial(
    pl.kernel,
    out_type=jax.ShapeDtypeStruct((NC, N), jnp.float32),
    mesh=_mesh,
    scratch_types=[
        pltpu.VMEM((1, C), jnp.int32),
        pltpu.VMEM((C,), jnp.float32),
        pltpu.VMEM_SHARED((N,), jnp.float32),
    ],
    compiler_params=pltpu.CompilerParams(use_tc_tiling_on_sc=False, needs_layout_passes=False),
)
def _deg_kernel(dst_hbm, ew_hbm, zeros1_hbm, out_hbm, dsti, ewv, acc):
    cid = lax.axis_index("c")
    sid = lax.axis_index("s")
    w = sid * NC + cid

    @pl.when(sid == 0)
    def _():
        pltpu.sync_copy(zeros1_hbm, acc)

    plsc.subcore_barrier()

    def body(t, carry):
        g = t * NW + w

        @pl.when(g < NCHUNK)
        def _():
            pltpu.sync_copy(dst_hbm.at[pl.ds(g * C, C)], dsti.at[0])
            pltpu.sync_copy(ew_hbm.at[pl.ds(g * C, C)], ewv)
            pltpu.sync_copy(ewv, acc.at[dsti.at[0]], add=True)

        return carry

    lax.fori_loop(0, TPW, body, 0)
    plsc.subcore_barrier()

    @pl.when(sid == 0)
    def _():
        pltpu.sync_copy(acc, out_hbm.at[cid])


@functools.partial(
    pl.kernel,
    out_type=jax.ShapeDtypeStruct((NC, N, D), jnp.float32),
    mesh=_mesh,
    scratch_types=[
        pltpu.VMEM((1, C), jnp.int32),
        pltpu.VMEM((1, C), jnp.int32),
        pltpu.VMEM((C,), jnp.float32),
        pltpu.VMEM((C, D), jnp.float32),
        pltpu.SemaphoreType.DMA,
        pltpu.VMEM_SHARED((N, D), jnp.float32),
    ],
    compiler_params=pltpu.CompilerParams(use_tc_tiling_on_sc=False, needs_layout_passes=False),
)
def _conv_kernel(src_hbm, dst_hbm, ew_hbm, y_hbm, zeros2_hbm, out_hbm,
                 srci, dsti, ewv, rows, sem, acc):
    cid = lax.axis_index("c")
    sid = lax.axis_index("s")
    w = sid * NC + cid

    pltpu.sync_copy(zeros2_hbm.at[pl.ds(sid * RPS, RPS)],
                    acc.at[pl.ds(sid * RPS, RPS)])
    plsc.subcore_barrier()

    def body(t, carry):
        g = t * NW + w

        @pl.when(g < NCHUNK)
        def _():
            pltpu.sync_copy(src_hbm.at[pl.ds(g * C, C)], srci.at[0])
            pltpu.sync_copy(dst_hbm.at[pl.ds(g * C, C)], dsti.at[0])
            pltpu.sync_copy(ew_hbm.at[pl.ds(g * C, C)], ewv)
            pltpu.async_copy(y_hbm.at[srci.at[0]], rows, sem).wait()

            def scale(e, c2):
                coef = plsc.load_gather(ewv, [jnp.full((L,), e, jnp.int32)])
                for r in range(D // L):
                    v = rows[e, pl.ds(r * L, L)]
                    rows[e, pl.ds(r * L, L)] = v * coef
                return c2

            lax.fori_loop(0, C, scale, 0)
            pltpu.sync_copy(rows, acc.at[dsti.at[0]], add=True)

        return carry

    lax.fori_loop(0, TPW, body, 0)
    plsc.subcore_barrier()
    pltpu.sync_copy(acc.at[pl.ds(sid * RPS, RPS)],
                    out_hbm.at[cid, pl.ds(sid * RPS, RPS)])


def _lstm_layer(x_seq, W_ih, W_hh, b_ih, b_hh):
    B = x_seq.shape[0]
    Hh = W_hh.shape[0]
    xs = jnp.transpose(x_seq, (1, 0, 2))

    def step(carry, xt):
        h, c = carry
        gates = xt @ W_ih + h @ W_hh + b_ih + b_hh
        i, f, g, o = jnp.split(gates, 4, axis=-1)
        i = jax.nn.sigmoid(i)
        f = jax.nn.sigmoid(f)
        g = jnp.tanh(g)
        o = jax.nn.sigmoid(o)
        c = f * c + i * g
        h = o * jnp.tanh(c)
        return (h, c), h

    init = (jnp.zeros((B, Hh), x_seq.dtype), jnp.zeros((B, Hh), x_seq.dtype))
    (_, _), ys = jax.lax.scan(step, init, xs)
    return jnp.transpose(ys, (1, 0, 2))


def kernel(x, edge_index, edge_weight, W_ih0, W_hh0, b_ih0, b_hh0,
           W_ih1, W_hh1, b_ih1, b_hh1, Wg0, bg0, Wg1, bg1,
           W_fc1, b_fc1, W_fc2, b_fc2):
    B, A, Sl, Ff = x.shape
    x_lstm = x.reshape(B * A, Sl, Ff)
    h0 = _lstm_layer(x_lstm, W_ih0, W_hh0, b_ih0, b_hh0)
    h1 = _lstm_layer(h0, W_ih1, W_hh1, b_ih1, b_hh1)
    h = h1[:, -1, :]

    src = edge_index[0]
    dst = edge_index[1]
    ew = edge_weight
    zeros1 = jnp.zeros((N,), jnp.float32)
    zeros2 = jnp.zeros((N, D), jnp.float32)

    degp = _deg_kernel(dst, ew, zeros1)
    deg = degp[0] + degp[1] + 1.0
    dinv = lax.rsqrt(deg)[:, None]

    y0 = dinv * (h @ Wg0)
    aggp0 = _conv_kernel(src, dst, ew, y0, zeros2)
    x1 = jax.nn.relu(dinv * (aggp0[0] + aggp0[1] + y0) + bg0)

    y1 = dinv * (x1 @ Wg1)
    aggp1 = _conv_kernel(src, dst, ew, y1, zeros2)
    g2 = dinv * (aggp1[0] + aggp1[1] + y1) + bg1

    out = jax.nn.relu(g2 @ W_fc1 + b_fc1) @ W_fc2 + b_fc2
    return out.reshape(B, A, -1)


# speedup vs baseline: 11.1312x; 1.1463x over previous
import functools

import jax
import jax.numpy as jnp
from jax import lax
from jax.experimental import pallas as pl
from jax.experimental.pallas import tpu as pltpu
from jax.experimental.pallas import tpu_sc as plsc

N = 10000
E = 640000
D = 64
NC = 2
NS = 16
NW = NC * NS
L = 16
C = 128
NCHUNK = E // C
TPW = -(-NCHUNK // NW)
RPS = N // NS

_mesh = plsc.VectorSubcoreMesh(
    core_axis_name="c", subcore_axis_name="s", num_cores=NC, num_subcores=NS)


@functools.partial(
    pl.kernel,
    out_type=jax.ShapeDtypeStruct((NC, N), jnp.float32),
    mesh=_mesh,
    scratch_types=[
        pltpu.VMEM((1, C), jnp.int32),
        pltpu.VMEM((C,), jnp.float32),
        pltpu.VMEM_SHARED((N,), jnp.float32),
    ],
    compiler_params=pltpu.CompilerParams(use_tc_tiling_on_sc=False, needs_layout_passes=False),
)
def _deg_kernel(dst_hbm, ew_hbm, zeros1_hbm, out_hbm, dsti, ewv, acc):
    cid = lax.axis_index("c")
    sid = lax.axis_index("s")
    w = sid * NC + cid

    @pl.when(sid == 0)
    def _():
        pltpu.sync_copy(zeros1_hbm, acc)

    plsc.subcore_barrier()

    def body(t, carry):
        g = t * NW + w

        @pl.when(g < NCHUNK)
        def _():
            pltpu.sync_copy(dst_hbm.at[pl.ds(g * C, C)], dsti.at[0])
            pltpu.sync_copy(ew_hbm.at[pl.ds(g * C, C)], ewv)
            pltpu.sync_copy(ewv, acc.at[dsti.at[0]], add=True)

        return carry

    lax.fori_loop(0, TPW, body, 0)
    plsc.subcore_barrier()

    @pl.when(sid == 0)
    def _():
        pltpu.sync_copy(acc, out_hbm.at[cid])


@functools.partial(
    pl.kernel,
    out_type=jax.ShapeDtypeStruct((NC, N, D), jnp.float32),
    mesh=_mesh,
    scratch_types=[
        pltpu.VMEM((1, C), jnp.int32),
        pltpu.VMEM((1, C), jnp.int32),
        pltpu.VMEM((C,), jnp.float32),
        pltpu.VMEM((C, D), jnp.float32),
        pltpu.SemaphoreType.DMA,
        pltpu.VMEM_SHARED((N, D), jnp.float32),
    ],
    compiler_params=pltpu.CompilerParams(use_tc_tiling_on_sc=False, needs_layout_passes=False),
)
def _conv_kernel(src_hbm, dst_hbm, ew_hbm, y_hbm, zeros2_hbm, out_hbm,
                 srci, dsti, ewv, rows, sem, acc):
    cid = lax.axis_index("c")
    sid = lax.axis_index("s")
    w = sid * NC + cid

    pltpu.sync_copy(zeros2_hbm.at[pl.ds(sid * RPS, RPS)],
                    acc.at[pl.ds(sid * RPS, RPS)])
    plsc.subcore_barrier()

    def body(t, carry):
        g = t * NW + w

        @pl.when(g < NCHUNK)
        def _():
            pltpu.sync_copy(src_hbm.at[pl.ds(g * C, C)], srci.at[0])
            pltpu.sync_copy(dst_hbm.at[pl.ds(g * C, C)], dsti.at[0])
            pltpu.sync_copy(ew_hbm.at[pl.ds(g * C, C)], ewv)
            pltpu.async_copy(y_hbm.at[srci.at[0]], rows, sem).wait()

            def scale(e, c2):
                coef = plsc.load_gather(ewv, [jnp.full((L,), e, jnp.int32)])
                for r in range(D // L):
                    v = rows[e, pl.ds(r * L, L)]
                    rows[e, pl.ds(r * L, L)] = v * coef
                return c2

            lax.fori_loop(0, C, scale, 0)
            pltpu.sync_copy(rows, acc.at[dsti.at[0]], add=True)

        return carry

    lax.fori_loop(0, TPW, body, 0)
    plsc.subcore_barrier()
    pltpu.sync_copy(acc.at[pl.ds(sid * RPS, RPS)],
                    out_hbm.at[cid, pl.ds(sid * RPS, RPS)])


R = 1000
S = 20
H = 64
G4 = 4 * H


def _lstm_body(xs_ref, wih0, whh0, b0, wih1, whh1, b1, wg0, deg2,
               y0_ref, dinv_ref, h0seq):
    def cell(gates, c):
        i = jax.nn.sigmoid(gates[:, 0:H])
        f = jax.nn.sigmoid(gates[:, H:2 * H])
        g = jnp.tanh(gates[:, 2 * H:3 * H])
        o = jax.nn.sigmoid(gates[:, 3 * H:4 * H])
        c = f * c + i * g
        h = o * jnp.tanh(c)
        return h, c

    z = jnp.zeros((R, H), jnp.float32)

    def step0(t, hc):
        h, c = hc
        gates = xs_ref[t] @ wih0[...] + h @ whh0[...] + b0[...]
        h, c = cell(gates, c)
        h0seq[t] = h
        return (h, c)

    lax.fori_loop(0, S, step0, (z, z))

    def step1(t, hc):
        h, c = hc
        gates = h0seq[t] @ wih1[...] + h @ whh1[...] + b1[...]
        return cell(gates, c)

    h1, _ = lax.fori_loop(0, S, step1, (z, z))

    dinv = lax.rsqrt(deg2[...])
    dinv_ref[...] = dinv
    y0_ref[...] = dinv * (h1 @ wg0[...])


def _lstm_kernel(xs, wih0, whh0, b0, wih1, whh1, b1, wg0, deg2):
    grid = N // R
    return pl.pallas_call(
        _lstm_body,
        grid=(grid,),
        in_specs=[
            pl.BlockSpec((S, R, 5), lambda i: (0, i, 0)),
            pl.BlockSpec((5, G4), lambda i: (0, 0)),
            pl.BlockSpec((H, G4), lambda i: (0, 0)),
            pl.BlockSpec((1, G4), lambda i: (0, 0)),
            pl.BlockSpec((H, G4), lambda i: (0, 0)),
            pl.BlockSpec((H, G4), lambda i: (0, 0)),
            pl.BlockSpec((1, G4), lambda i: (0, 0)),
            pl.BlockSpec((H, D), lambda i: (0, 0)),
            pl.BlockSpec((R, 1), lambda i: (i, 0)),
        ],
        out_specs=[
            pl.BlockSpec((R, D), lambda i: (i, 0)),
            pl.BlockSpec((R, 1), lambda i: (i, 0)),
        ],
        out_shape=[
            jax.ShapeDtypeStruct((N, D), jnp.float32),
            jax.ShapeDtypeStruct((N, 1), jnp.float32),
        ],
        scratch_shapes=[pltpu.VMEM((S, R, H), jnp.float32)],
    )(xs, wih0, whh0, b0, wih1, whh1, b1, wg0, deg2)


def _mid_body(aggp, y0, dinv, bg0, wg1, y1_ref):
    agg = aggp[0] + aggp[1] + y0[...]
    x1 = jax.nn.relu(dinv[...] * agg + bg0[...])
    y1_ref[...] = dinv[...] * (x1 @ wg1[...])


def _mid_kernel(aggp, y0, dinv, bg0, wg1):
    grid = N // R
    return pl.pallas_call(
        _mid_body,
        grid=(grid,),
        in_specs=[
            pl.BlockSpec((2, R, D), lambda i: (0, i, 0)),
            pl.BlockSpec((R, D), lambda i: (i, 0)),
            pl.BlockSpec((R, 1), lambda i: (i, 0)),
            pl.BlockSpec((1, D), lambda i: (0, 0)),
            pl.BlockSpec((D, D), lambda i: (0, 0)),
        ],
        out_specs=pl.BlockSpec((R, D), lambda i: (i, 0)),
        out_shape=jax.ShapeDtypeStruct((N, D), jnp.float32),
    )(aggp, y0, dinv, bg0, wg1)


def _head_body(aggp, y1, dinv, bg1, wfc1, bfc1, wfc2, bfc2, out_ref):
    g2 = dinv[...] * (aggp[0] + aggp[1] + y1[...]) + bg1[...]
    hf = jax.nn.relu(g2 @ wfc1[...] + bfc1[...])
    out_ref[...] = hf @ wfc2[...] + bfc2[...]


def _head_kernel(aggp, y1, dinv, bg1, wfc1, bfc1, wfc2, bfc2):
    grid = N // R
    return pl.pallas_call(
        _head_body,
        grid=(grid,),
        in_specs=[
            pl.BlockSpec((2, R, D), lambda i: (0, i, 0)),
            pl.BlockSpec((R, D), lambda i: (i, 0)),
            pl.BlockSpec((R, 1), lambda i: (i, 0)),
            pl.BlockSpec((1, D), lambda i: (0, 0)),
            pl.BlockSpec((D, D // 2), lambda i: (0, 0)),
            pl.BlockSpec((1, D // 2), lambda i: (0, 0)),
            pl.BlockSpec((D // 2, 1), lambda i: (0, 0)),
            pl.BlockSpec((1, 1), lambda i: (0, 0)),
        ],
        out_specs=pl.BlockSpec((R, 1), lambda i: (i, 0)),
        out_shape=jax.ShapeDtypeStruct((N, 1), jnp.float32),
    )(aggp, y1, dinv, bg1, wfc1, bfc1, wfc2, bfc2)


def kernel(x, edge_index, edge_weight, W_ih0, W_hh0, b_ih0, b_hh0,
           W_ih1, W_hh1, b_ih1, b_hh1, Wg0, bg0, Wg1, bg1,
           W_fc1, b_fc1, W_fc2, b_fc2):
    B, A, Sl, Ff = x.shape
    xs = jnp.transpose(x.reshape(B * A, Sl, Ff), (1, 0, 2))

    src = edge_index[0]
    dst = edge_index[1]
    ew = edge_weight
    zeros1 = jnp.zeros((N,), jnp.float32)
    zeros2 = jnp.zeros((N, D), jnp.float32)

    degp = _deg_kernel(dst, ew, zeros1)
    deg2 = (degp[0] + degp[1] + 1.0)[:, None]

    b0 = (b_ih0 + b_hh0)[None, :]
    b1 = (b_ih1 + b_hh1)[None, :]
    y0, dinv = _lstm_kernel(xs, W_ih0, W_hh0, b0, W_ih1, W_hh1, b1, Wg0, deg2)

    aggp0 = _conv_kernel(src, dst, ew, y0, zeros2)
    y1 = _mid_kernel(aggp0, y0, dinv, bg0[None, :], Wg1)

    aggp1 = _conv_kernel(src, dst, ew, y1, zeros2)
    out = _head_kernel(aggp1, y1, dinv, bg1[None, :], W_fc1,
                       b_fc1[None, :], W_fc2, b_fc2[None, :])
    return out.reshape(B, A, -1)


# speedup vs baseline: 20.8030x; 1.8689x over previous
import functools

import jax
import jax.numpy as jnp
from jax import lax
from jax.experimental import pallas as pl
from jax.experimental.pallas import tpu as pltpu
from jax.experimental.pallas import tpu_sc as plsc

N = 10000
E = 640000
D = 64
NC = 2
NS = 16
NW = NC * NS
L = 16
C = 128
NCHUNK = E // C
SK = 4
NSUP = NCHUNK // SK
TSUP = -(-NSUP // NW)
RPS = N // NS

_mesh = plsc.VectorSubcoreMesh(
    core_axis_name="c", subcore_axis_name="s", num_cores=NC, num_subcores=NS)


_sc_params = pltpu.CompilerParams(
    use_tc_tiling_on_sc=False, needs_layout_passes=False)


@functools.partial(
    pl.kernel,
    out_type=jax.ShapeDtypeStruct((NC, N), jnp.float32),
    mesh=_mesh,
    scratch_types=[
        pltpu.VMEM((2, SK, C), jnp.int32),
        pltpu.VMEM((2, SK, C), jnp.float32),
        pltpu.SemaphoreType.DMA((2,)),
        pltpu.SemaphoreType.DMA((2, SK)),
        pltpu.VMEM_SHARED((N,), jnp.float32),
    ],
    compiler_params=_sc_params,
)
def _deg_kernel(dst_hbm, ew_hbm, zeros1_hbm, out_hbm, dsti, ewv, sem_i, sem_s, acc):
    cid = lax.axis_index("c")
    sid = lax.axis_index("s")
    w = sid * NC + cid

    @pl.when(sid == 0)
    def _():
        pltpu.sync_copy(zeros1_hbm, acc)

    plsc.subcore_barrier()

    def load_idx(t, b):
        sup = t * NW + w

        @pl.when(sup < NSUP)
        def _():
            pltpu.async_copy(dst_hbm.at[pl.ds(sup * SK, SK)], dsti.at[b],
                             sem_i.at[b])
            pltpu.async_copy(ew_hbm.at[pl.ds(sup * SK, SK)], ewv.at[b],
                             sem_i.at[b])

    def drain(t, b):
        sup = t * NW + w

        @pl.when((t >= 0) & (sup < NSUP))
        def _():
            for k in range(SK):
                pltpu.make_async_copy(
                    ewv.at[b, k], acc.at[dsti.at[b, k]], sem_s.at[b, k]).wait()

    def sup_body(t, b):
        sup = t * NW + w
        drain(t - 2, b)
        load_idx(t + 1, 1 - b)

        @pl.when(sup < NSUP)
        def _():
            pltpu.make_async_copy(dst_hbm.at[pl.ds(sup * SK, SK)], dsti.at[b],
                                  sem_i.at[b]).wait()
            pltpu.make_async_copy(ew_hbm.at[pl.ds(sup * SK, SK)], ewv.at[b],
                                  sem_i.at[b]).wait()
            for k in range(SK):
                pltpu.async_copy(ewv.at[b, k], acc.at[dsti.at[b, k]],
                                 sem_s.at[b, k], add=True)

    load_idx(0, 0)

    def body(tt, carry):
        sup_body(tt * 2, 0)
        sup_body(tt * 2 + 1, 1)
        return carry

    lax.fori_loop(0, TSUP // 2, body, 0)
    drain(TSUP - 2, (TSUP - 2) % 2)
    drain(TSUP - 1, (TSUP - 1) % 2)
    plsc.subcore_barrier()

    @pl.when(sid == 0)
    def _():
        pltpu.sync_copy(acc, out_hbm.at[cid])


@functools.partial(
    pl.kernel,
    out_type=jax.ShapeDtypeStruct((NC, N, D), jnp.float32),
    mesh=_mesh,
    scratch_types=[
        pltpu.VMEM((2, SK, C), jnp.int32),
        pltpu.VMEM((2, SK, C), jnp.int32),
        pltpu.VMEM((2, SK, C), jnp.float32),
        pltpu.VMEM((2, SK, C, D), jnp.float32),
        pltpu.SemaphoreType.DMA((2,)),
        pltpu.SemaphoreType.DMA((SK,)),
        pltpu.SemaphoreType.DMA((2, SK)),
        pltpu.VMEM_SHARED((N, D), jnp.float32),
    ],
    compiler_params=_sc_params,
)
def _conv_kernel(src_hbm, dst_hbm, ew_hbm, y_hbm, zeros2_hbm, out_hbm,
                 srci, dsti, ewv, rows, sem_i, sem_g, sem_s, acc):
    cid = lax.axis_index("c")
    sid = lax.axis_index("s")
    w = sid * NC + cid

    pltpu.sync_copy(zeros2_hbm.at[pl.ds(sid * RPS, RPS)],
                    acc.at[pl.ds(sid * RPS, RPS)])
    plsc.subcore_barrier()

    def load_idx(t, b):
        sup = t * NW + w

        @pl.when(sup < NSUP)
        def _():
            pltpu.async_copy(src_hbm.at[pl.ds(sup * SK, SK)], srci.at[b],
                             sem_i.at[b])
            pltpu.async_copy(dst_hbm.at[pl.ds(sup * SK, SK)], dsti.at[b],
                             sem_i.at[b])
            pltpu.async_copy(ew_hbm.at[pl.ds(sup * SK, SK)], ewv.at[b],
                             sem_i.at[b])

    def drain(t, b):
        sup = t * NW + w

        @pl.when((t >= 0) & (sup < NSUP))
        def _():
            for k in range(SK):
                pltpu.make_async_copy(
                    rows.at[b, k], acc.at[dsti.at[b, k]], sem_s.at[b, k]).wait()

    def scale_chunk(b, k):
        rbk = rows.at[b, k]
        ewk = ewv.at[b, k]

        def sc(e, cr):
            coef = plsc.load_gather(ewk, [jnp.full((L,), e, jnp.int32)])
            for r in range(D // L):
                rbk[e, pl.ds(r * L, L)] = rbk[e, pl.ds(r * L, L)] * coef
            return cr

        lax.fori_loop(0, C, sc, 0, unroll=8)

    def sup_body(t, b):
        sup = t * NW + w
        drain(t - 2, b)
        load_idx(t + 1, 1 - b)

        @pl.when(sup < NSUP)
        def _():
            pltpu.make_async_copy(src_hbm.at[pl.ds(sup * SK, SK)], srci.at[b],
                                  sem_i.at[b]).wait()
            pltpu.make_async_copy(dst_hbm.at[pl.ds(sup * SK, SK)], dsti.at[b],
                                  sem_i.at[b]).wait()
            pltpu.make_async_copy(ew_hbm.at[pl.ds(sup * SK, SK)], ewv.at[b],
                                  sem_i.at[b]).wait()
            for k in range(SK):
                pltpu.async_copy(y_hbm.at[srci.at[b, k]], rows.at[b, k],
                                 sem_g.at[k])
            for k in range(SK):
                pltpu.make_async_copy(y_hbm.at[srci.at[b, k]], rows.at[b, k],
                                      sem_g.at[k]).wait()
                scale_chunk(b, k)
                pltpu.async_copy(rows.at[b, k], acc.at[dsti.at[b, k]],
                                 sem_s.at[b, k], add=True)

    load_idx(0, 0)

    def body(tt, carry):
        sup_body(tt * 2, 0)
        sup_body(tt * 2 + 1, 1)
        return carry

    lax.fori_loop(0, TSUP // 2, body, 0)
    drain(TSUP - 2, (TSUP - 2) % 2)
    drain(TSUP - 1, (TSUP - 1) % 2)
    plsc.subcore_barrier()
    pltpu.sync_copy(acc.at[pl.ds(sid * RPS, RPS)],
                    out_hbm.at[cid, pl.ds(sid * RPS, RPS)])


R = 1000
S = 20
H = 64
G4 = 4 * H


def _lstm_body(xs_ref, wih0, whh0, b0, wih1, whh1, b1, wg0, deg2,
               y0_ref, dinv_ref, h0seq):
    def cell(gates, c):
        i = jax.nn.sigmoid(gates[:, 0:H])
        f = jax.nn.sigmoid(gates[:, H:2 * H])
        g = jnp.tanh(gates[:, 2 * H:3 * H])
        o = jax.nn.sigmoid(gates[:, 3 * H:4 * H])
        c = f * c + i * g
        h = o * jnp.tanh(c)
        return h, c

    z = jnp.zeros((R, H), jnp.float32)

    def step0(t, hc):
        h, c = hc
        gates = xs_ref[t] @ wih0[...] + h @ whh0[...] + b0[...]
        h, c = cell(gates, c)
        h0seq[t] = h
        return (h, c)

    lax.fori_loop(0, S, step0, (z, z))

    def step1(t, hc):
        h, c = hc
        gates = h0seq[t] @ wih1[...] + h @ whh1[...] + b1[...]
        return cell(gates, c)

    h1, _ = lax.fori_loop(0, S, step1, (z, z))

    dinv = lax.rsqrt(deg2[...])
    dinv_ref[...] = dinv
    y0_ref[...] = dinv * (h1 @ wg0[...])


def _lstm_kernel(xs, wih0, whh0, b0, wih1, whh1, b1, wg0, deg2):
    grid = N // R
    return pl.pallas_call(
        _lstm_body,
        grid=(grid,),
        in_specs=[
            pl.BlockSpec((S, R, 5), lambda i: (0, i, 0)),
            pl.BlockSpec((5, G4), lambda i: (0, 0)),
            pl.BlockSpec((H, G4), lambda i: (0, 0)),
            pl.BlockSpec((1, G4), lambda i: (0, 0)),
            pl.BlockSpec((H, G4), lambda i: (0, 0)),
            pl.BlockSpec((H, G4), lambda i: (0, 0)),
            pl.BlockSpec((1, G4), lambda i: (0, 0)),
            pl.BlockSpec((H, D), lambda i: (0, 0)),
            pl.BlockSpec((R, 1), lambda i: (i, 0)),
        ],
        out_specs=[
            pl.BlockSpec((R, D), lambda i: (i, 0)),
            pl.BlockSpec((R, 1), lambda i: (i, 0)),
        ],
        out_shape=[
            jax.ShapeDtypeStruct((N, D), jnp.float32),
            jax.ShapeDtypeStruct((N, 1), jnp.float32),
        ],
        scratch_shapes=[pltpu.VMEM((S, R, H), jnp.float32)],
    )(xs, wih0, whh0, b0, wih1, whh1, b1, wg0, deg2)


def _mid_body(aggp, y0, dinv, bg0, wg1, y1_ref):
    agg = aggp[0] + aggp[1] + y0[...]
    x1 = jax.nn.relu(dinv[...] * agg + bg0[...])
    y1_ref[...] = dinv[...] * (x1 @ wg1[...])


def _mid_kernel(aggp, y0, dinv, bg0, wg1):
    grid = N // R
    return pl.pallas_call(
        _mid_body,
        grid=(grid,),
        in_specs=[
            pl.BlockSpec((2, R, D), lambda i: (0, i, 0)),
            pl.BlockSpec((R, D), lambda i: (i, 0)),
            pl.BlockSpec((R, 1), lambda i: (i, 0)),
            pl.BlockSpec((1, D), lambda i: (0, 0)),
            pl.BlockSpec((D, D), lambda i: (0, 0)),
        ],
        out_specs=pl.BlockSpec((R, D), lambda i: (i, 0)),
        out_shape=jax.ShapeDtypeStruct((N, D), jnp.float32),
    )(aggp, y0, dinv, bg0, wg1)


def _head_body(aggp, y1, dinv, bg1, wfc1, bfc1, wfc2, bfc2, out_ref):
    g2 = dinv[...] * (aggp[0] + aggp[1] + y1[...]) + bg1[...]
    hf = jax.nn.relu(g2 @ wfc1[...] + bfc1[...])
    out_ref[...] = hf @ wfc2[...] + bfc2[...]


def _head_kernel(aggp, y1, dinv, bg1, wfc1, bfc1, wfc2, bfc2):
    grid = N // R
    return pl.pallas_call(
        _head_body,
        grid=(grid,),
        in_specs=[
            pl.BlockSpec((2, R, D), lambda i: (0, i, 0)),
            pl.BlockSpec((R, D), lambda i: (i, 0)),
            pl.BlockSpec((R, 1), lambda i: (i, 0)),
            pl.BlockSpec((1, D), lambda i: (0, 0)),
            pl.BlockSpec((D, D // 2), lambda i: (0, 0)),
            pl.BlockSpec((1, D // 2), lambda i: (0, 0)),
            pl.BlockSpec((D // 2, 1), lambda i: (0, 0)),
            pl.BlockSpec((1, 1), lambda i: (0, 0)),
        ],
        out_specs=pl.BlockSpec((R, 1), lambda i: (i, 0)),
        out_shape=jax.ShapeDtypeStruct((N, 1), jnp.float32),
    )(aggp, y1, dinv, bg1, wfc1, bfc1, wfc2, bfc2)


def kernel(x, edge_index, edge_weight, W_ih0, W_hh0, b_ih0, b_hh0,
           W_ih1, W_hh1, b_ih1, b_hh1, Wg0, bg0, Wg1, bg1,
           W_fc1, b_fc1, W_fc2, b_fc2):
    B, A, Sl, Ff = x.shape
    xs = jnp.transpose(x.reshape(B * A, Sl, Ff), (1, 0, 2))

    src = edge_index[0].reshape(NCHUNK, C)
    dst = edge_index[1].reshape(NCHUNK, C)
    ew = edge_weight.reshape(NCHUNK, C)
    zeros1 = jnp.zeros((N,), jnp.float32)
    zeros2 = jnp.zeros((N, D), jnp.float32)

    degp = _deg_kernel(dst, ew, zeros1)
    deg2 = (degp[0] + degp[1] + 1.0)[:, None]

    b0 = (b_ih0 + b_hh0)[None, :]
    b1 = (b_ih1 + b_hh1)[None, :]
    y0, dinv = _lstm_kernel(xs, W_ih0, W_hh0, b0, W_ih1, W_hh1, b1, Wg0, deg2)

    aggp0 = _conv_kernel(src, dst, ew, y0, zeros2)
    y1 = _mid_kernel(aggp0, y0, dinv, bg0[None, :], Wg1)

    aggp1 = _conv_kernel(src, dst, ew, y1, zeros2)
    out = _head_kernel(aggp1, y1, dinv, bg1[None, :], W_fc1,
                       b_fc1[None, :], W_fc2, b_fc2[None, :])
    return out.reshape(B, A, -1)


# speedup vs baseline: 21.6421x; 1.0403x over previous
import functools

import jax
import jax.numpy as jnp
from jax import lax
from jax.experimental import pallas as pl
from jax.experimental.pallas import tpu as pltpu
from jax.experimental.pallas import tpu_sc as plsc

N = 10000
E = 640000
D = 64
NC = 2
NS = 16
NW = NC * NS
L = 16
C = 128
NCHUNK = E // C
SK = 4
NSUP = NCHUNK // SK
TSUP = -(-NSUP // NW)
RPS = N // NS

_mesh = plsc.VectorSubcoreMesh(
    core_axis_name="c", subcore_axis_name="s", num_cores=NC, num_subcores=NS)


_sc_params = pltpu.CompilerParams(
    use_tc_tiling_on_sc=False, needs_layout_passes=False)


@functools.partial(
    pl.kernel,
    out_type=jax.ShapeDtypeStruct((NC, N), jnp.float32),
    mesh=_mesh,
    scratch_types=[
        pltpu.VMEM((2, SK, C), jnp.int32),
        pltpu.VMEM((2, SK, C), jnp.float32),
        pltpu.SemaphoreType.DMA((2,)),
        pltpu.SemaphoreType.DMA((2, SK)),
        pltpu.VMEM_SHARED((N,), jnp.float32),
    ],
    compiler_params=_sc_params,
)
def _deg_kernel(dst_hbm, ew_hbm, zeros1_hbm, out_hbm, dsti, ewv, sem_i, sem_s, acc):
    cid = lax.axis_index("c")
    sid = lax.axis_index("s")
    w = sid * NC + cid

    @pl.when(sid == 0)
    def _():
        pltpu.sync_copy(zeros1_hbm, acc)

    plsc.subcore_barrier()

    def load_idx(t, b):
        sup = t * NW + w

        @pl.when(sup < NSUP)
        def _():
            pltpu.async_copy(dst_hbm.at[pl.ds(sup * SK, SK)], dsti.at[b],
                             sem_i.at[b])
            pltpu.async_copy(ew_hbm.at[pl.ds(sup * SK, SK)], ewv.at[b],
                             sem_i.at[b])

    def drain(t, b):
        sup = t * NW + w

        @pl.when((t >= 0) & (sup < NSUP))
        def _():
            for k in range(SK):
                pltpu.make_async_copy(
                    ewv.at[b, k], acc.at[dsti.at[b, k]], sem_s.at[b, k]).wait()

    def sup_body(t, b):
        sup = t * NW + w
        drain(t - 2, b)
        load_idx(t + 1, 1 - b)

        @pl.when(sup < NSUP)
        def _():
            pltpu.make_async_copy(dst_hbm.at[pl.ds(sup * SK, SK)], dsti.at[b],
                                  sem_i.at[b]).wait()
            pltpu.make_async_copy(ew_hbm.at[pl.ds(sup * SK, SK)], ewv.at[b],
                                  sem_i.at[b]).wait()
            for k in range(SK):
                pltpu.async_copy(ewv.at[b, k], acc.at[dsti.at[b, k]],
                                 sem_s.at[b, k], add=True)

    load_idx(0, 0)

    def body(tt, carry):
        sup_body(tt * 2, 0)
        sup_body(tt * 2 + 1, 1)
        return carry

    lax.fori_loop(0, TSUP // 2, body, 0)
    drain(TSUP - 2, (TSUP - 2) % 2)
    drain(TSUP - 1, (TSUP - 1) % 2)
    plsc.subcore_barrier()

    @pl.when(sid == 0)
    def _():
        pltpu.sync_copy(acc, out_hbm.at[cid])


@functools.partial(
    pl.kernel,
    out_type=jax.ShapeDtypeStruct((NC, N, D), jnp.float32),
    mesh=_mesh,
    scratch_types=[
        pltpu.VMEM((2, SK, C), jnp.int32),
        pltpu.VMEM((2, SK, C), jnp.int32),
        pltpu.VMEM((2, SK, C), jnp.float32),
        pltpu.VMEM((2, SK, C, D), jnp.float32),
        pltpu.SemaphoreType.DMA((2,)),
        pltpu.SemaphoreType.DMA((SK,)),
        pltpu.SemaphoreType.DMA((2, SK)),
        pltpu.VMEM_SHARED((N, D), jnp.float32),
    ],
    compiler_params=_sc_params,
)
def _conv_kernel(src_hbm, dst_hbm, ew_hbm, y_hbm, zeros2_hbm, out_hbm,
                 srci, dsti, ewv, rows, sem_i, sem_g, sem_s, acc):
    cid = lax.axis_index("c")
    sid = lax.axis_index("s")
    w = sid * NC + cid

    pltpu.sync_copy(zeros2_hbm.at[pl.ds(sid * RPS, RPS)],
                    acc.at[pl.ds(sid * RPS, RPS)])
    plsc.subcore_barrier()

    def load_idx(t, b):
        sup = t * NW + w

        @pl.when(sup < NSUP)
        def _():
            pltpu.async_copy(src_hbm.at[pl.ds(sup * SK, SK)], srci.at[b],
                             sem_i.at[b])
            pltpu.async_copy(dst_hbm.at[pl.ds(sup * SK, SK)], dsti.at[b],
                             sem_i.at[b])
            pltpu.async_copy(ew_hbm.at[pl.ds(sup * SK, SK)], ewv.at[b],
                             sem_i.at[b])

    def drain(t, b):
        sup = t * NW + w

        @pl.when((t >= 0) & (sup < NSUP))
        def _():
            for k in range(SK):
                pltpu.make_async_copy(
                    rows.at[b, k], acc.at[dsti.at[b, k]], sem_s.at[b, k]).wait()

    def scale_chunk(b, k):
        rbk = rows.at[b, k]
        ewk = ewv.at[b, k]

        def sc(e, cr):
            coef = plsc.load_gather(ewk, [jnp.full((L,), e, jnp.int32)])
            for r in range(D // L):
                rbk[e, pl.ds(r * L, L)] = rbk[e, pl.ds(r * L, L)] * coef
            return cr

        lax.fori_loop(0, C, sc, 0, unroll=8)

    def sup_body(t, b):
        sup = t * NW + w
        drain(t - 2, b)
        load_idx(t + 1, 1 - b)

        @pl.when(sup < NSUP)
        def _():
            pltpu.make_async_copy(src_hbm.at[pl.ds(sup * SK, SK)], srci.at[b],
                                  sem_i.at[b]).wait()
            pltpu.make_async_copy(dst_hbm.at[pl.ds(sup * SK, SK)], dsti.at[b],
                                  sem_i.at[b]).wait()
            pltpu.make_async_copy(ew_hbm.at[pl.ds(sup * SK, SK)], ewv.at[b],
                                  sem_i.at[b]).wait()
            for k in range(SK):
                pltpu.async_copy(y_hbm.at[srci.at[b, k]], rows.at[b, k],
                                 sem_g.at[k])
            for k in range(SK):
                pltpu.make_async_copy(y_hbm.at[srci.at[b, k]], rows.at[b, k],
                                      sem_g.at[k]).wait()
                scale_chunk(b, k)
                pltpu.async_copy(rows.at[b, k], acc.at[dsti.at[b, k]],
                                 sem_s.at[b, k], add=True)

    load_idx(0, 0)

    def body(tt, carry):
        sup_body(tt * 2, 0)
        sup_body(tt * 2 + 1, 1)
        return carry

    lax.fori_loop(0, TSUP // 2, body, 0)
    drain(TSUP - 2, (TSUP - 2) % 2)
    drain(TSUP - 1, (TSUP - 1) % 2)
    plsc.subcore_barrier()
    pltpu.sync_copy(acc.at[pl.ds(sid * RPS, RPS)],
                    out_hbm.at[cid, pl.ds(sid * RPS, RPS)])


R = 2000
S = 20
H = 64
G4 = 4 * H


def _lstm_body(xs_ref, wih0, whh0, b0, wih1, whh1, b1, wg0, deg2,
               y0_ref, dinv_ref):
    def cell(gates, c):
        i = jax.nn.sigmoid(gates[:, 0:H])
        f = jax.nn.sigmoid(gates[:, H:2 * H])
        g = jnp.tanh(gates[:, 2 * H:3 * H])
        o = jax.nn.sigmoid(gates[:, 3 * H:4 * H])
        c = f * c + i * g
        h = o * jnp.tanh(c)
        return h, c

    z = jnp.zeros((R, H), jnp.float32)

    def step(t, hc):
        h0, c0, h1, c1 = hc
        g0 = xs_ref[t] @ wih0[...] + h0 @ whh0[...] + b0[...]
        h0, c0 = cell(g0, c0)
        g1 = h0 @ wih1[...] + h1 @ whh1[...] + b1[...]
        h1, c1 = cell(g1, c1)
        return (h0, c0, h1, c1)

    _, _, h1, _ = lax.fori_loop(0, S, step, (z, z, z, z))

    dinv = lax.rsqrt(deg2[...])
    dinv_ref[...] = dinv
    y0_ref[...] = dinv * (h1 @ wg0[...])


def _lstm_kernel(xs, wih0, whh0, b0, wih1, whh1, b1, wg0, deg2):
    grid = N // R
    return pl.pallas_call(
        _lstm_body,
        grid=(grid,),
        in_specs=[
            pl.BlockSpec((S, R, 5), lambda i: (0, i, 0)),
            pl.BlockSpec((5, G4), lambda i: (0, 0)),
            pl.BlockSpec((H, G4), lambda i: (0, 0)),
            pl.BlockSpec((1, G4), lambda i: (0, 0)),
            pl.BlockSpec((H, G4), lambda i: (0, 0)),
            pl.BlockSpec((H, G4), lambda i: (0, 0)),
            pl.BlockSpec((1, G4), lambda i: (0, 0)),
            pl.BlockSpec((H, D), lambda i: (0, 0)),
            pl.BlockSpec((R, 1), lambda i: (i, 0)),
        ],
        out_specs=[
            pl.BlockSpec((R, D), lambda i: (i, 0)),
            pl.BlockSpec((R, 1), lambda i: (i, 0)),
        ],
        out_shape=[
            jax.ShapeDtypeStruct((N, D), jnp.float32),
            jax.ShapeDtypeStruct((N, 1), jnp.float32),
        ],
    )(xs, wih0, whh0, b0, wih1, whh1, b1, wg0, deg2)


def _mid_body(aggp, y0, dinv, bg0, wg1, y1_ref):
    agg = aggp[0] + aggp[1] + y0[...]
    x1 = jax.nn.relu(dinv[...] * agg + bg0[...])
    y1_ref[...] = dinv[...] * (x1 @ wg1[...])


def _mid_kernel(aggp, y0, dinv, bg0, wg1):
    grid = N // R
    return pl.pallas_call(
        _mid_body,
        grid=(grid,),
        in_specs=[
            pl.BlockSpec((2, R, D), lambda i: (0, i, 0)),
            pl.BlockSpec((R, D), lambda i: (i, 0)),
            pl.BlockSpec((R, 1), lambda i: (i, 0)),
            pl.BlockSpec((1, D), lambda i: (0, 0)),
            pl.BlockSpec((D, D), lambda i: (0, 0)),
        ],
        out_specs=pl.BlockSpec((R, D), lambda i: (i, 0)),
        out_shape=jax.ShapeDtypeStruct((N, D), jnp.float32),
    )(aggp, y0, dinv, bg0, wg1)


def _head_body(aggp, y1, dinv, bg1, wfc1, bfc1, wfc2, bfc2, out_ref):
    g2 = dinv[...] * (aggp[0] + aggp[1] + y1[...]) + bg1[...]
    hf = jax.nn.relu(g2 @ wfc1[...] + bfc1[...])
    out_ref[...] = hf @ wfc2[...] + bfc2[...]


def _head_kernel(aggp, y1, dinv, bg1, wfc1, bfc1, wfc2, bfc2):
    grid = N // R
    return pl.pallas_call(
        _head_body,
        grid=(grid,),
        in_specs=[
            pl.BlockSpec((2, R, D), lambda i: (0, i, 0)),
            pl.BlockSpec((R, D), lambda i: (i, 0)),
            pl.BlockSpec((R, 1), lambda i: (i, 0)),
            pl.BlockSpec((1, D), lambda i: (0, 0)),
            pl.BlockSpec((D, D // 2), lambda i: (0, 0)),
            pl.BlockSpec((1, D // 2), lambda i: (0, 0)),
            pl.BlockSpec((D // 2, 1), lambda i: (0, 0)),
            pl.BlockSpec((1, 1), lambda i: (0, 0)),
        ],
        out_specs=pl.BlockSpec((R, 1), lambda i: (i, 0)),
        out_shape=jax.ShapeDtypeStruct((N, 1), jnp.float32),
    )(aggp, y1, dinv, bg1, wfc1, bfc1, wfc2, bfc2)


def kernel(x, edge_index, edge_weight, W_ih0, W_hh0, b_ih0, b_hh0,
           W_ih1, W_hh1, b_ih1, b_hh1, Wg0, bg0, Wg1, bg1,
           W_fc1, b_fc1, W_fc2, b_fc2):
    B, A, Sl, Ff = x.shape
    xs = jnp.transpose(x.reshape(B * A, Sl, Ff), (1, 0, 2))

    src = edge_index[0].reshape(NCHUNK, C)
    dst = edge_index[1].reshape(NCHUNK, C)
    ew = edge_weight.reshape(NCHUNK, C)
    zeros1 = jnp.zeros((N,), jnp.float32)
    zeros2 = jnp.zeros((N, D), jnp.float32)

    degp = _deg_kernel(dst, ew, zeros1)
    deg2 = (degp[0] + degp[1] + 1.0)[:, None]

    b0 = (b_ih0 + b_hh0)[None, :]
    b1 = (b_ih1 + b_hh1)[None, :]
    y0, dinv = _lstm_kernel(xs, W_ih0, W_hh0, b0, W_ih1, W_hh1, b1, Wg0, deg2)

    aggp0 = _conv_kernel(src, dst, ew, y0, zeros2)
    y1 = _mid_kernel(aggp0, y0, dinv, bg0[None, :], Wg1)

    aggp1 = _conv_kernel(src, dst, ew, y1, zeros2)
    out = _head_kernel(aggp1, y1, dinv, bg1[None, :], W_fc1,
                       b_fc1[None, :], W_fc2, b_fc2[None, :])
    return out.reshape(B, A, -1)
